# baseline (speedup 1.0000x reference)
import functools

import jax
import jax.numpy as jnp
import numpy as np
from jax import lax
from jax.experimental import pallas as pl
from jax.experimental.pallas import tpu as pltpu
from jax.experimental.pallas import tpu_sc as plsc

EMBED = 512
_L = 16


def _make_pe(d_model, max_len):
    position = np.arange(max_len, dtype=np.float32)[:, None]
    div_term = np.exp(
        np.arange(0, d_model, 2, dtype=np.float32) * (-np.log(10000.0) / d_model)
    )
    pe = np.zeros((max_len, d_model), dtype=np.float32)
    pe[:, 0::2] = np.sin(position * div_term)
    pe[:, 1::2] = np.cos(position * div_term)
    return pe


def _sc_scan(ap_hbm, tp_hbm, out_hbm, ap_v, tp_v, idx_v, B, T_text, T_frame):
    r = lax.axis_index("s")

    pltpu.sync_copy(ap_hbm.at[pl.ds(r * T_frame, T_frame)], ap_v.at[pl.ds(0, T_frame)])
    pltpu.sync_copy(tp_hbm.at[pl.ds(r * T_text, T_text)], tp_v)
    lanes = lax.iota(jnp.int32, _L)
    ap_v[pl.ds(T_frame, _L)] = jnp.zeros((_L,), jnp.int32)
    idx_v[pl.ds(0, _L)] = jnp.zeros((_L,), jnp.float32)

    before0 = plsc.load_gather(tp_v, [jnp.zeros((_L,), jnp.int32)])
    cap = jnp.full((_L,), T_text - 1, jnp.int32)

    def cond(carry):
        f, _, _ = carry
        return f < T_frame

    def body(carry):
        f, p, before = carry
        av = ap_v[pl.ds(f, _L)]
        neq = av != before
        kv = plsc.all_reduce_ffs(neq)
        adv = kv < _L
        p_new = jnp.where(adv, p + 1, p)
        safe_new = jnp.minimum(p_new, cap)
        tpv = plsc.load_gather(tp_v, [safe_new])
        before_new = jnp.where(adv, tpv, before)
        safe_old_f = jnp.minimum(p, cap).astype(jnp.float32)
        val = jnp.where(lanes < kv, safe_old_f, safe_new.astype(jnp.float32))
        kcap = jnp.minimum(kv, _L - 1)
        plsc.store_scatter(
            idx_v, [f + lanes], val, mask=lanes <= kcap
        )
        k_s = jnp.min(kcap)
        return (f + k_s + 1, p_new, before_new)

    lax.while_loop(cond, body, (jnp.int32(1), jnp.zeros((_L,), jnp.int32), before0))
    pltpu.sync_copy(idx_v.at[pl.ds(0, T_frame)], out_hbm.at[pl.ds(r * T_frame, T_frame)])


def _postnet_kernel(
    enc_ref,
    idx_ref,
    pitch_ref,
    beats_ref,
    wp_ref,
    demb_ref,
    wt_ref,
    bias_ref,
    pe_ref,
    out_ref,
    pew_scr,
):
    b = pl.program_id(0)
    T_frame, B = idx_ref.shape
    T_text = enc_ref.shape[1]

    @pl.when(b == 0)
    def _prologue():
        pew_scr[...] = (
            jnp.dot(pe_ref[...], wt_ref[...], preferred_element_type=jnp.float32)
            + bias_ref[...]
        )

    bhot = (
        jax.lax.broadcasted_iota(jnp.int32, (B, 1), 0) == b
    ).astype(jnp.float32)
    idx_col = jnp.dot(idx_ref[...], bhot, preferred_element_type=jnp.float32)
    pitch_col = jnp.dot(pitch_ref[...], bhot, preferred_element_type=jnp.float32)
    beats_col = jnp.dot(beats_ref[...], bhot, preferred_element_type=jnp.float32)

    enc = enc_ref[0]
    g = enc.astype(jnp.float32) + jnp.dot(
        enc, wt_ref[...], preferred_element_type=jnp.float32
    )
    oh = (
        jax.lax.broadcasted_iota(jnp.int32, (T_frame, T_text), 1)
        == idx_col.astype(jnp.int32)
    ).astype(jnp.bfloat16)
    gathered = jnp.dot(
        oh, g.astype(jnp.bfloat16), preferred_element_type=jnp.float32
    )

    out_ref[0] = (
        gathered
        + pitch_col * wp_ref[...]
        + beats_col * demb_ref[...]
        + pew_scr[...]
    )


@jax.jit
def kernel(
    encoder_out,
    align_phone,
    text_phone,
    pitch,
    beats,
    fc_pitch_w,
    fc_pitch_b,
    fc_pos_w,
    fc_pos_b,
    emb_beats,
):
    B, T_text, D = encoder_out.shape
    T_frame = align_phone.shape[1]

    ap_flat = align_phone.astype(jnp.int32).reshape(B * T_frame)
    tp_flat = text_phone.astype(jnp.int32).reshape(B * T_text)
    mesh = plsc.VectorSubcoreMesh(
        core_axis_name="c", subcore_axis_name="s", num_cores=1
    )
    scan = functools.partial(
        pl.kernel,
        mesh=mesh,
        out_type=jax.ShapeDtypeStruct((B * T_frame,), jnp.float32),
        scratch_types=[
            pltpu.VMEM((T_frame + _L,), jnp.int32),
            pltpu.VMEM((T_text,), jnp.int32),
            pltpu.VMEM((T_frame + _L,), jnp.float32),
        ],
        compiler_params=pltpu.CompilerParams(needs_layout_passes=False),
    )(functools.partial(_sc_scan, B=B, T_text=T_text, T_frame=T_frame))
    idx_t = scan(ap_flat, tp_flat).reshape(B, T_frame).T

    pitch_t = jnp.squeeze(pitch, -1).T
    beats_t = jnp.squeeze(beats, -1).astype(jnp.float32).T
    wp = fc_pitch_w.reshape(1, D)
    wt = fc_pos_w.T.astype(jnp.bfloat16)
    bias = (fc_pos_b + fc_pitch_b + emb_beats[0]).reshape(1, D)
    demb = (emb_beats[1] - emb_beats[0]).reshape(1, D)
    pe = jnp.asarray(_make_pe(D, T_frame)).astype(jnp.bfloat16)

    out = pl.pallas_call(
        _postnet_kernel,
        grid=(B,),
        in_specs=[
            pl.BlockSpec((1, T_text, D), lambda b: (b, 0, 0)),
            pl.BlockSpec((T_frame, B), lambda b: (0, 0)),
            pl.BlockSpec((T_frame, B), lambda b: (0, 0)),
            pl.BlockSpec((T_frame, B), lambda b: (0, 0)),
            pl.BlockSpec((1, D), lambda b: (0, 0)),
            pl.BlockSpec((1, D), lambda b: (0, 0)),
            pl.BlockSpec((D, D), lambda b: (0, 0)),
            pl.BlockSpec((1, D), lambda b: (0, 0)),
            pl.BlockSpec((T_frame, D), lambda b: (0, 0)),
        ],
        out_specs=pl.BlockSpec((1, T_frame, D), lambda b: (b, 0, 0)),
        out_shape=jax.ShapeDtypeStruct((B, T_frame, D), jnp.float32),
        scratch_shapes=[
            pltpu.VMEM((T_frame, D), jnp.float32),
        ],
        compiler_params=pltpu.CompilerParams(
            dimension_semantics=("arbitrary",),
        ),
    )(
        encoder_out.astype(jnp.bfloat16),
        idx_t,
        pitch_t,
        beats_t,
        wp,
        demb,
        wt,
        bias,
        pe,
    )
    return out


# speedup vs baseline: 1.1115x; 1.1115x over previous
import functools

import jax
import jax.numpy as jnp
import numpy as np
from jax import lax
from jax.experimental import pallas as pl
from jax.experimental.pallas import tpu as pltpu
from jax.experimental.pallas import tpu_sc as plsc

EMBED = 512
_L = 16


def _make_pe(d_model, max_len):
    position = np.arange(max_len, dtype=np.float32)[:, None]
    div_term = np.exp(
        np.arange(0, d_model, 2, dtype=np.float32) * (-np.log(10000.0) / d_model)
    )
    pe = np.zeros((max_len, d_model), dtype=np.float32)
    pe[:, 0::2] = np.sin(position * div_term)
    pe[:, 1::2] = np.cos(position * div_term)
    return pe


def _sc_scan(ap_hbm, tp_hbm, out_hbm, ap_v, tp_v, idx_v, B, T_text, T_frame):
    r = lax.axis_index("s")

    pltpu.sync_copy(ap_hbm.at[pl.ds(r * T_frame, T_frame)], ap_v.at[pl.ds(0, T_frame)])
    pltpu.sync_copy(tp_hbm.at[pl.ds(r * T_text, T_text)], tp_v)
    lanes = lax.iota(jnp.int32, _L)
    ap_v[pl.ds(T_frame, _L)] = jnp.zeros((_L,), jnp.int32)
    idx_v[pl.ds(0, _L)] = jnp.zeros((_L,), jnp.float32)

    before0 = plsc.load_gather(tp_v, [jnp.zeros((_L,), jnp.int32)])
    cap = jnp.full((_L,), T_text - 1, jnp.int32)

    def cond(carry):
        f, _, _ = carry
        return f < T_frame

    def body(carry):
        f, p, before = carry
        av = ap_v[pl.ds(f, _L)]
        neq = av != before
        kv = plsc.all_reduce_ffs(neq)
        adv = kv < _L
        p_new = jnp.where(adv, p + 1, p)
        safe_new = jnp.minimum(p_new, cap)
        tpv = plsc.load_gather(tp_v, [safe_new])
        before_new = jnp.where(adv, tpv, before)
        safe_old_f = jnp.minimum(p, cap).astype(jnp.float32)
        val = jnp.where(lanes < kv, safe_old_f, safe_new.astype(jnp.float32))
        kcap = jnp.minimum(kv, _L - 1)
        plsc.store_scatter(
            idx_v, [f + lanes], val, mask=lanes <= kcap
        )
        k_s = jnp.min(kcap)
        return (f + k_s + 1, p_new, before_new)

    lax.while_loop(cond, body, (jnp.int32(1), jnp.zeros((_L,), jnp.int32), before0))
    pltpu.sync_copy(idx_v.at[pl.ds(0, T_frame)], out_hbm.at[pl.ds(r * T_frame, T_frame)])


def _postnet_kernel(
    enc_ref,
    idx_ref,
    pitch_ref,
    beats_ref,
    wp_ref,
    bp_ref,
    wt_ref,
    bpos_ref,
    emb_ref,
    pe_ref,
    out_ref,
    pew_scr,
    idxt_scr,
):
    b = pl.program_id(0)
    B, T_frame = idx_ref.shape
    T_text = enc_ref.shape[1]

    @pl.when(b == 0)
    def _prologue():
        pew_scr[...] = (
            jnp.dot(pe_ref[...], wt_ref[...], preferred_element_type=jnp.float32)
            + bpos_ref[...]
            + bp_ref[...]
            + emb_ref[0:1, :]
        )
        idxt_scr[...] = jnp.transpose(idx_ref[...], (1, 0))

    bhot = (
        jax.lax.broadcasted_iota(jnp.int32, (B, 1), 0) == b
    ).astype(jnp.float32)
    idx_col = jnp.dot(idxt_scr[...], bhot, preferred_element_type=jnp.float32)
    pitch_col = jnp.dot(pitch_ref[...], bhot, preferred_element_type=jnp.float32)
    beats_col = jnp.dot(beats_ref[...], bhot, preferred_element_type=jnp.float32)

    enc = enc_ref[0]
    g = enc + jnp.dot(
        enc.astype(jnp.bfloat16), wt_ref[...], preferred_element_type=jnp.float32
    )
    oh = (
        jax.lax.broadcasted_iota(jnp.int32, (T_frame, T_text), 1)
        == idx_col.astype(jnp.int32)
    ).astype(jnp.bfloat16)
    gathered = jnp.dot(
        oh, g.astype(jnp.bfloat16), preferred_element_type=jnp.float32
    )

    demb = emb_ref[1:2, :] - emb_ref[0:1, :]
    out_ref[0] = (
        gathered
        + pitch_col * wp_ref[...]
        + beats_col * demb
        + pew_scr[...]
    )


@jax.jit
def kernel(
    encoder_out,
    align_phone,
    text_phone,
    pitch,
    beats,
    fc_pitch_w,
    fc_pitch_b,
    fc_pos_w,
    fc_pos_b,
    emb_beats,
):
    B, T_text, D = encoder_out.shape
    T_frame = align_phone.shape[1]

    ap_flat = align_phone.astype(jnp.int32).reshape(B * T_frame)
    tp_flat = text_phone.astype(jnp.int32).reshape(B * T_text)
    mesh = plsc.VectorSubcoreMesh(
        core_axis_name="c", subcore_axis_name="s", num_cores=1
    )
    scan = functools.partial(
        pl.kernel,
        mesh=mesh,
        out_type=jax.ShapeDtypeStruct((B * T_frame,), jnp.float32),
        scratch_types=[
            pltpu.VMEM((T_frame + _L,), jnp.int32),
            pltpu.VMEM((T_text,), jnp.int32),
            pltpu.VMEM((T_frame + _L,), jnp.float32),
        ],
        compiler_params=pltpu.CompilerParams(needs_layout_passes=False),
    )(functools.partial(_sc_scan, B=B, T_text=T_text, T_frame=T_frame))
    idx_nat = scan(ap_flat, tp_flat).reshape(B, T_frame)

    pitch_t = jnp.squeeze(pitch, -1).T
    beats_t = jnp.squeeze(beats, -1).astype(jnp.float32).T
    wp = fc_pitch_w.reshape(1, D)
    bp = fc_pitch_b.reshape(1, D)
    wt = fc_pos_w.T.astype(jnp.bfloat16)
    bpos = fc_pos_b.reshape(1, D)
    pe = jnp.asarray(_make_pe(D, T_frame)).astype(jnp.bfloat16)

    out = pl.pallas_call(
        _postnet_kernel,
        grid=(B,),
        in_specs=[
            pl.BlockSpec((1, T_text, D), lambda b: (b, 0, 0)),
            pl.BlockSpec((B, T_frame), lambda b: (0, 0)),
            pl.BlockSpec((T_frame, B), lambda b: (0, 0)),
            pl.BlockSpec((T_frame, B), lambda b: (0, 0)),
            pl.BlockSpec((1, D), lambda b: (0, 0)),
            pl.BlockSpec((1, D), lambda b: (0, 0)),
            pl.BlockSpec((D, D), lambda b: (0, 0)),
            pl.BlockSpec((1, D), lambda b: (0, 0)),
            pl.BlockSpec((2, D), lambda b: (0, 0)),
            pl.BlockSpec((T_frame, D), lambda b: (0, 0)),
        ],
        out_specs=pl.BlockSpec((1, T_frame, D), lambda b: (b, 0, 0)),
        out_shape=jax.ShapeDtypeStruct((B, T_frame, D), jnp.float32),
        scratch_shapes=[
            pltpu.VMEM((T_frame, D), jnp.float32),
            pltpu.VMEM((T_frame, B), jnp.float32),
        ],
        compiler_params=pltpu.CompilerParams(
            dimension_semantics=("arbitrary",),
        ),
    )(
        encoder_out,
        idx_nat,
        pitch_t,
        beats_t,
        wp,
        bp,
        wt,
        bpos,
        emb_beats,
        pe,
    )
    return out


# speedup vs baseline: 1.1273x; 1.0142x over previous
import functools

import jax
import jax.numpy as jnp
import numpy as np
from jax import lax
from jax.experimental import pallas as pl
from jax.experimental.pallas import tpu as pltpu
from jax.experimental.pallas import tpu_sc as plsc

EMBED = 512
_L = 16


def _make_pe(d_model, max_len):
    position = np.arange(max_len, dtype=np.float32)[:, None]
    div_term = np.exp(
        np.arange(0, d_model, 2, dtype=np.float32) * (-np.log(10000.0) / d_model)
    )
    pe = np.zeros((max_len, d_model), dtype=np.float32)
    pe[:, 0::2] = np.sin(position * div_term)
    pe[:, 1::2] = np.cos(position * div_term)
    return pe


def _sc_scan(ap_hbm, tp_hbm, out_hbm, ap_v, tp_v, idx_v, B, T_text, T_frame):
    r = lax.axis_index("s")

    pltpu.sync_copy(ap_hbm.at[pl.ds(r * T_frame, T_frame)], ap_v.at[pl.ds(0, T_frame)])
    pltpu.sync_copy(tp_hbm.at[pl.ds(r * T_text, T_text)], tp_v)
    lanes = lax.iota(jnp.int32, _L)
    ap_v[pl.ds(T_frame, _L)] = jnp.zeros((_L,), jnp.int32)
    idx_v[pl.ds(0, _L)] = jnp.zeros((_L,), jnp.float32)

    before0 = plsc.load_gather(tp_v, [jnp.zeros((_L,), jnp.int32)])
    cap = jnp.full((_L,), T_text - 1, jnp.int32)

    def cond(carry):
        f, _, _ = carry
        return f < T_frame

    def body(carry):
        f, p, before = carry
        av = ap_v[pl.ds(f, _L)]
        neq = av != before
        kv = plsc.all_reduce_ffs(neq)
        adv = kv < _L
        p_new = jnp.where(adv, p + 1, p)
        safe_new = jnp.minimum(p_new, cap)
        tpv = plsc.load_gather(tp_v, [safe_new])
        before_new = jnp.where(adv, tpv, before)
        safe_old_f = jnp.minimum(p, cap).astype(jnp.float32)
        val = jnp.where(lanes < kv, safe_old_f, safe_new.astype(jnp.float32))
        kcap = jnp.minimum(kv, _L - 1)
        plsc.store_scatter(
            idx_v, [f + lanes], val, mask=lanes <= kcap
        )
        k_s = jnp.min(kcap)
        return (f + k_s + 1, p_new, before_new)

    lax.while_loop(cond, body, (jnp.int32(1), jnp.zeros((_L,), jnp.int32), before0))
    pltpu.sync_copy(idx_v.at[pl.ds(0, T_frame)], out_hbm.at[pl.ds(r * T_frame, T_frame)])


def _postnet_kernel(
    enc_ref,
    idx_ref,
    pitch_ref,
    beats_ref,
    wp_ref,
    bp_ref,
    wt_ref,
    bpos_ref,
    emb_ref,
    pe_ref,
    out_ref,
    pew_scr,
    idxt_scr,
    pitcht_scr,
    beatst_scr,
):
    b = pl.program_id(0)
    B, T_frame = idx_ref.shape
    T_text = enc_ref.shape[1]

    @pl.when(b == 0)
    def _prologue():
        pew_scr[...] = (
            jnp.dot(pe_ref[...], wt_ref[...], preferred_element_type=jnp.float32)
            + bpos_ref[...]
            + bp_ref[...]
            + emb_ref[0:1, :]
        )
        idxt_scr[...] = jnp.transpose(idx_ref[...], (1, 0))
        pitcht_scr[...] = jnp.transpose(pitch_ref[...], (1, 0))
        beatst_scr[...] = jnp.transpose(beats_ref[...], (1, 0))

    bhot = (
        jax.lax.broadcasted_iota(jnp.int32, (B, 1), 0) == b
    ).astype(jnp.float32)
    idx_col = jnp.dot(idxt_scr[...], bhot, preferred_element_type=jnp.float32)
    pitch_col = jnp.dot(pitcht_scr[...], bhot, preferred_element_type=jnp.float32)
    beats_col = jnp.dot(beatst_scr[...], bhot, preferred_element_type=jnp.float32)

    enc = enc_ref[0]
    g = enc + jnp.dot(
        enc.astype(jnp.bfloat16), wt_ref[...], preferred_element_type=jnp.float32
    )
    oh = (
        jax.lax.broadcasted_iota(jnp.int32, (T_frame, T_text), 1)
        == idx_col.astype(jnp.int32)
    ).astype(jnp.bfloat16)
    gathered = jnp.dot(
        oh, g.astype(jnp.bfloat16), preferred_element_type=jnp.float32
    )

    demb = emb_ref[1:2, :] - emb_ref[0:1, :]
    out_ref[0] = (
        gathered
        + pitch_col * wp_ref[...]
        + beats_col * demb
        + pew_scr[...]
    )


@jax.jit
def kernel(
    encoder_out,
    align_phone,
    text_phone,
    pitch,
    beats,
    fc_pitch_w,
    fc_pitch_b,
    fc_pos_w,
    fc_pos_b,
    emb_beats,
):
    B, T_text, D = encoder_out.shape
    T_frame = align_phone.shape[1]

    ap_flat = align_phone.astype(jnp.int32).reshape(B * T_frame)
    tp_flat = text_phone.astype(jnp.int32).reshape(B * T_text)
    mesh = plsc.VectorSubcoreMesh(
        core_axis_name="c", subcore_axis_name="s", num_cores=1
    )
    scan = functools.partial(
        pl.kernel,
        mesh=mesh,
        out_type=jax.ShapeDtypeStruct((B * T_frame,), jnp.float32),
        scratch_types=[
            pltpu.VMEM((T_frame + _L,), jnp.int32),
            pltpu.VMEM((T_text,), jnp.int32),
            pltpu.VMEM((T_frame + _L,), jnp.float32),
        ],
        compiler_params=pltpu.CompilerParams(needs_layout_passes=False),
    )(functools.partial(_sc_scan, B=B, T_text=T_text, T_frame=T_frame))
    idx_nat = scan(ap_flat, tp_flat).reshape(B, T_frame)

    pitch_nat = jnp.squeeze(pitch, -1)
    beats_nat = jnp.squeeze(beats, -1).astype(jnp.float32)
    wp = fc_pitch_w.reshape(1, D)
    bp = fc_pitch_b.reshape(1, D)
    wt = fc_pos_w.T.astype(jnp.bfloat16)
    bpos = fc_pos_b.reshape(1, D)
    pe = jnp.asarray(_make_pe(D, T_frame)).astype(jnp.bfloat16)

    out = pl.pallas_call(
        _postnet_kernel,
        grid=(B,),
        in_specs=[
            pl.BlockSpec((1, T_text, D), lambda b: (b, 0, 0)),
            pl.BlockSpec((B, T_frame), lambda b: (0, 0)),
            pl.BlockSpec((B, T_frame), lambda b: (0, 0)),
            pl.BlockSpec((B, T_frame), lambda b: (0, 0)),
            pl.BlockSpec((1, D), lambda b: (0, 0)),
            pl.BlockSpec((1, D), lambda b: (0, 0)),
            pl.BlockSpec((D, D), lambda b: (0, 0)),
            pl.BlockSpec((1, D), lambda b: (0, 0)),
            pl.BlockSpec((2, D), lambda b: (0, 0)),
            pl.BlockSpec((T_frame, D), lambda b: (0, 0)),
        ],
        out_specs=pl.BlockSpec((1, T_frame, D), lambda b: (b, 0, 0)),
        out_shape=jax.ShapeDtypeStruct((B, T_frame, D), jnp.float32),
        scratch_shapes=[
            pltpu.VMEM((T_frame, D), jnp.float32),
            pltpu.VMEM((T_frame, B), jnp.float32),
            pltpu.VMEM((T_frame, B), jnp.float32),
            pltpu.VMEM((T_frame, B), jnp.float32),
        ],
        compiler_params=pltpu.CompilerParams(
            dimension_semantics=("arbitrary",),
        ),
    )(
        encoder_out,
        idx_nat,
        pitch_nat,
        beats_nat,
        wp,
        bp,
        wt,
        bpos,
        emb_beats,
        pe,
    )
    return out


# speedup vs baseline: 1.1796x; 1.0464x over previous
import functools

import jax
import jax.numpy as jnp
import numpy as np
from jax import lax
from jax.experimental import pallas as pl
from jax.experimental.pallas import tpu as pltpu
from jax.experimental.pallas import tpu_sc as plsc

EMBED = 512
_L = 16


def _make_pe(d_model, max_len):
    position = np.arange(max_len, dtype=np.float32)[:, None]
    div_term = np.exp(
        np.arange(0, d_model, 2, dtype=np.float32) * (-np.log(10000.0) / d_model)
    )
    pe = np.zeros((max_len, d_model), dtype=np.float32)
    pe[:, 0::2] = np.sin(position * div_term)
    pe[:, 1::2] = np.cos(position * div_term)
    return pe


def _sc_scan(ap_hbm, tp_hbm, out_hbm, ap_v, tp_v, idx_v, B, T_text, T_frame):
    r = lax.axis_index("s")

    pltpu.sync_copy(ap_hbm.at[r], ap_v.at[pl.ds(0, T_frame)])
    pltpu.sync_copy(tp_hbm.at[r], tp_v)
    lanes = lax.iota(jnp.int32, _L)
    ap_v[pl.ds(T_frame, _L)] = jnp.zeros((_L,), jnp.int32)
    idx_v[pl.ds(0, _L)] = jnp.zeros((_L,), jnp.float32)

    before0 = plsc.load_gather(tp_v, [jnp.zeros((_L,), jnp.int32)])
    cap = jnp.full((_L,), T_text - 1, jnp.int32)

    def cond(carry):
        f, _, _ = carry
        return f < T_frame

    def body(carry):
        f, p, before = carry
        av = ap_v[pl.ds(f, _L)]
        neq = av != before
        kv = plsc.all_reduce_ffs(neq)
        adv = kv < _L
        p_new = jnp.where(adv, p + 1, p)
        safe_new = jnp.minimum(p_new, cap)
        tpv = plsc.load_gather(tp_v, [safe_new])
        before_new = jnp.where(adv, tpv, before)
        safe_old_f = jnp.minimum(p, cap).astype(jnp.float32)
        val = jnp.where(lanes < kv, safe_old_f, safe_new.astype(jnp.float32))
        kcap = jnp.minimum(kv, _L - 1)
        plsc.store_scatter(
            idx_v, [f + lanes], val, mask=lanes <= kcap
        )
        k_s = jnp.min(kcap)
        return (f + k_s + 1, p_new, before_new)

    lax.while_loop(cond, body, (jnp.int32(1), jnp.zeros((_L,), jnp.int32), before0))
    pltpu.sync_copy(idx_v.at[pl.ds(0, T_frame)], out_hbm.at[r])


def _postnet_kernel(
    enc_ref,
    idx_ref,
    pitch_ref,
    beats_ref,
    wp_ref,
    bp_ref,
    w_ref,
    bpos_ref,
    emb_ref,
    pe_ref,
    out_ref,
    pew_scr,
    idxt_scr,
    pitcht_scr,
    beatst_scr,
    wt_scr,
):
    b = pl.program_id(0)
    B, T_frame = idx_ref.shape
    T_text = enc_ref.shape[1]

    @pl.when(b == 0)
    def _prologue():
        wt_scr[...] = jnp.transpose(w_ref[...], (1, 0)).astype(jnp.bfloat16)
        pew_scr[...] = (
            jnp.dot(pe_ref[...], wt_scr[...], preferred_element_type=jnp.float32)
            + bpos_ref[...]
            + bp_ref[...]
            + emb_ref[0:1, :]
        )
        idxt_scr[...] = jnp.transpose(idx_ref[...], (1, 0))
        pitcht_scr[...] = jnp.transpose(pitch_ref[...], (1, 0))
        beatst_scr[...] = jnp.transpose(
            beats_ref[...].astype(jnp.float32), (1, 0)
        )

    bhot = (
        jax.lax.broadcasted_iota(jnp.int32, (B, 1), 0) == b
    ).astype(jnp.float32)
    idx_col = jnp.dot(idxt_scr[...], bhot, preferred_element_type=jnp.float32)
    pitch_col = jnp.dot(pitcht_scr[...], bhot, preferred_element_type=jnp.float32)
    beats_col = jnp.dot(beatst_scr[...], bhot, preferred_element_type=jnp.float32)

    enc = enc_ref[0]
    g = enc + jnp.dot(
        enc.astype(jnp.bfloat16), wt_scr[...], preferred_element_type=jnp.float32
    )
    oh = (
        jax.lax.broadcasted_iota(jnp.int32, (T_frame, T_text), 1)
        == idx_col.astype(jnp.int32)
    ).astype(jnp.bfloat16)
    gathered = jnp.dot(
        oh, g.astype(jnp.bfloat16), preferred_element_type=jnp.float32
    )

    demb = emb_ref[1:2, :] - emb_ref[0:1, :]
    out_ref[0] = (
        gathered
        + pitch_col * wp_ref[...]
        + beats_col * demb
        + pew_scr[...]
    )


@jax.jit
def kernel(
    encoder_out,
    align_phone,
    text_phone,
    pitch,
    beats,
    fc_pitch_w,
    fc_pitch_b,
    fc_pos_w,
    fc_pos_b,
    emb_beats,
):
    B, T_text, D = encoder_out.shape
    T_frame = align_phone.shape[1]

    mesh = plsc.VectorSubcoreMesh(
        core_axis_name="c", subcore_axis_name="s", num_cores=1
    )
    scan = functools.partial(
        pl.kernel,
        mesh=mesh,
        out_type=jax.ShapeDtypeStruct((B, T_frame), jnp.float32),
        scratch_types=[
            pltpu.VMEM((T_frame + _L,), jnp.int32),
            pltpu.VMEM((T_text,), jnp.int32),
            pltpu.VMEM((T_frame + _L,), jnp.float32),
        ],
        compiler_params=pltpu.CompilerParams(needs_layout_passes=False),
    )(functools.partial(_sc_scan, B=B, T_text=T_text, T_frame=T_frame))
    idx_nat = scan(align_phone.astype(jnp.int32), text_phone.astype(jnp.int32))

    pitch_nat = jnp.squeeze(pitch, -1)
    beats_nat = jnp.squeeze(beats, -1)
    wp = fc_pitch_w.reshape(1, D)
    bp = fc_pitch_b.reshape(1, D)
    bpos = fc_pos_b.reshape(1, D)
    pe = jnp.asarray(_make_pe(D, T_frame)).astype(jnp.bfloat16)

    out = pl.pallas_call(
        _postnet_kernel,
        grid=(B,),
        in_specs=[
            pl.BlockSpec((1, T_text, D), lambda b: (b, 0, 0)),
            pl.BlockSpec((B, T_frame), lambda b: (0, 0)),
            pl.BlockSpec((B, T_frame), lambda b: (0, 0)),
            pl.BlockSpec((B, T_frame), lambda b: (0, 0)),
            pl.BlockSpec((1, D), lambda b: (0, 0)),
            pl.BlockSpec((1, D), lambda b: (0, 0)),
            pl.BlockSpec((D, D), lambda b: (0, 0)),
            pl.BlockSpec((1, D), lambda b: (0, 0)),
            pl.BlockSpec((2, D), lambda b: (0, 0)),
            pl.BlockSpec((T_frame, D), lambda b: (0, 0)),
        ],
        out_specs=pl.BlockSpec((1, T_frame, D), lambda b: (b, 0, 0)),
        out_shape=jax.ShapeDtypeStruct((B, T_frame, D), jnp.float32),
        scratch_shapes=[
            pltpu.VMEM((T_frame, D), jnp.float32),
            pltpu.VMEM((T_frame, B), jnp.float32),
            pltpu.VMEM((T_frame, B), jnp.float32),
            pltpu.VMEM((T_frame, B), jnp.float32),
            pltpu.VMEM((D, D), jnp.bfloat16),
        ],
        compiler_params=pltpu.CompilerParams(
            dimension_semantics=("arbitrary",),
        ),
    )(
        encoder_out,
        idx_nat,
        pitch_nat,
        beats_nat,
        wp,
        bp,
        fc_pos_w,
        bpos,
        emb_beats,
        pe,
    )
    return out
